# speedup vs baseline: 1.5444x; 1.5444x over previous
import functools

import jax
import jax.numpy as jnp
from jax import lax
from jax.experimental import pallas as pl
from jax.experimental.pallas import tpu as pltpu
from jax.experimental.pallas import tpu_sc as plsc

_NUM_CORES = 2
_NUM_SUBCORES = 16
_NUM_WORKERS = _NUM_CORES * _NUM_SUBCORES


def _make_gather(n_rows: int, d: int, chunk: int):
  rows_per_w = n_rows // _NUM_WORKERS
  n_iters = rows_per_w // chunk
  mesh = plsc.VectorSubcoreMesh(core_axis_name="c", subcore_axis_name="s")

  @functools.partial(
      pl.kernel,
      out_type=jax.ShapeDtypeStruct((n_rows, d), jnp.float32),
      mesh=mesh,
      scratch_types=[
          pltpu.VMEM((n_iters, chunk), jnp.int32),
          pltpu.VMEM((chunk, d), jnp.float32),
          pltpu.SemaphoreType.DMA,
      ],
  )
  def gather_kernel(idx_hbm, table_hbm, out_hbm, idx_v, buf, sem):
    wid = lax.axis_index("s") * _NUM_CORES + lax.axis_index("c")
    base = wid * rows_per_w
    pltpu.sync_copy(idx_hbm.at[wid], idx_v)

    def body(i, carry):
      pltpu.async_copy(table_hbm.at[idx_v.at[i]], buf, sem).wait()
      pltpu.sync_copy(buf, out_hbm.at[pl.ds(base + i * chunk, chunk)])
      return carry

    lax.fori_loop(0, n_iters, body, 0)

  return gather_kernel


def kernel(prefix, emb_table):
  b, s = prefix.shape
  _, d = emb_table.shape
  n = b * s
  chunk = 2
  rows_per_w = n // _NUM_WORKERS
  idx = prefix.reshape(_NUM_WORKERS, rows_per_w // chunk, chunk)
  out = _make_gather(n, d, chunk)(idx, emb_table)
  return out.reshape(b, s, d)


# speedup vs baseline: 1.7868x; 1.1570x over previous
import functools

import jax
import jax.numpy as jnp
from jax import lax
from jax.experimental import pallas as pl
from jax.experimental.pallas import tpu as pltpu
from jax.experimental.pallas import tpu_sc as plsc

_NUM_CORES = 2
_NUM_SUBCORES = 16
_NUM_WORKERS = _NUM_CORES * _NUM_SUBCORES


def _make_gather(n_rows: int, d: int, chunk: int, nbuf: int):
  rows_per_w = n_rows // _NUM_WORKERS
  n_iters = rows_per_w // chunk
  n_rounds = n_iters // nbuf
  mesh = plsc.VectorSubcoreMesh(core_axis_name="c", subcore_axis_name="s")

  @functools.partial(
      pl.kernel,
      out_type=jax.ShapeDtypeStruct((n_rows, d), jnp.float32),
      mesh=mesh,
      scratch_types=[
          pltpu.VMEM((n_iters, chunk), jnp.int32),
          [pltpu.VMEM((chunk, d), jnp.float32) for _ in range(nbuf)],
          [pltpu.SemaphoreType.DMA for _ in range(nbuf)],
          [pltpu.SemaphoreType.DMA for _ in range(nbuf)],
      ],
  )
  def gather_kernel(idx_hbm, table_hbm, out_hbm, idx_v, bufs, gsems, wsems):
    wid = lax.axis_index("s") * _NUM_CORES + lax.axis_index("c")
    base = wid * rows_per_w
    pltpu.sync_copy(idx_hbm.at[wid], idx_v)

    for b in range(nbuf):
      pltpu.async_copy(table_hbm.at[idx_v.at[b]], bufs[b], gsems[b])

    def out_slice(i):
      return out_hbm.at[pl.ds(base + i * chunk, chunk)]

    def round_body(j, carry):
      i0 = j * nbuf
      for b in range(nbuf):
        pltpu.make_async_copy(table_hbm.at[idx_v.at[i0 + b]], bufs[b],
                              gsems[b]).wait()
        pltpu.async_copy(bufs[b], out_slice(i0 + b), wsems[b])
      for b in range(nbuf):
        pltpu.make_async_copy(bufs[b], out_slice(i0 + b), wsems[b]).wait()

        @pl.when(i0 + b + nbuf < n_iters)
        def _():
          pltpu.async_copy(table_hbm.at[idx_v.at[i0 + b + nbuf]], bufs[b],
                           gsems[b])

      return carry

    lax.fori_loop(0, n_rounds, round_body, 0)

  return gather_kernel


def kernel(prefix, emb_table):
  b, s = prefix.shape
  _, d = emb_table.shape
  n = b * s
  chunk = 1
  nbuf = 4
  rows_per_w = n // _NUM_WORKERS
  idx = prefix.reshape(_NUM_WORKERS, rows_per_w // chunk, chunk)
  out = _make_gather(n, d, chunk, nbuf)(idx, emb_table)
  return out.reshape(b, s, d)


# speedup vs baseline: 1.8011x; 1.0080x over previous
import functools

import jax
import jax.numpy as jnp
from jax import lax
from jax.experimental import pallas as pl
from jax.experimental.pallas import tpu as pltpu
from jax.experimental.pallas import tpu_sc as plsc

_NUM_CORES = 2
_NUM_SUBCORES = 16
_NUM_WORKERS = _NUM_CORES * _NUM_SUBCORES


def _make_gather(n_rows: int, d: int, chunk: int, nbuf: int):
  rows_per_w = n_rows // _NUM_WORKERS
  n_iters = rows_per_w // chunk
  n_rounds = n_iters // nbuf
  mesh = plsc.VectorSubcoreMesh(core_axis_name="c", subcore_axis_name="s")

  @functools.partial(
      pl.kernel,
      out_type=jax.ShapeDtypeStruct((n_rows, d), jnp.float32),
      mesh=mesh,
      scratch_types=[
          pltpu.VMEM((n_iters, chunk), jnp.int32),
          [pltpu.VMEM((chunk, d), jnp.float32) for _ in range(nbuf)],
          [pltpu.SemaphoreType.DMA for _ in range(nbuf)],
          [pltpu.SemaphoreType.DMA for _ in range(nbuf)],
      ],
  )
  def gather_kernel(idx_hbm, table_hbm, out_hbm, idx_v, bufs, gsems, wsems):
    wid = lax.axis_index("s") * _NUM_CORES + lax.axis_index("c")
    base = wid * rows_per_w
    pltpu.sync_copy(idx_hbm.at[wid], idx_v)

    lag = nbuf // 2

    def out_slice(i):
      return out_hbm.at[pl.ds(base + i * chunk, chunk)]

    def start_gather(i, b):
      pltpu.async_copy(table_hbm.at[idx_v.at[i]], bufs[b], gsems[b])

    def wait_gather(i, b):
      pltpu.make_async_copy(table_hbm.at[idx_v.at[i]], bufs[b],
                            gsems[b]).wait()

    def start_write(i, b):
      pltpu.async_copy(bufs[b], out_slice(i), wsems[b])

    def wait_write(i, b):
      pltpu.make_async_copy(bufs[b], out_slice(i), wsems[b]).wait()

    for b in range(nbuf):
      start_gather(b, b)
      if b >= lag:
        bw = b - lag
        wait_gather(bw, bw)
        start_write(bw, bw)

    def round_body(j, carry):
      i0 = j * nbuf
      for b in range(nbuf):
        ig = i0 + b
        wait_write(ig - nbuf, b)
        start_gather(ig, b)
        bw = (b - lag) % nbuf
        iw = ig - lag
        wait_gather(iw, bw)
        start_write(iw, bw)
      return carry

    lax.fori_loop(1, n_rounds, round_body, 0)

    for t in range(lag):
      iw = n_iters - lag + t
      bw = iw % nbuf
      wait_gather(iw, bw)
      start_write(iw, bw)
    for b in range(nbuf):
      wait_write(n_iters - nbuf + b, b)

  return gather_kernel


def kernel(prefix, emb_table):
  b, s = prefix.shape
  _, d = emb_table.shape
  n = b * s
  chunk = 1
  nbuf = 4
  rows_per_w = n // _NUM_WORKERS
  idx = prefix.reshape(_NUM_WORKERS, rows_per_w // chunk, chunk)
  out = _make_gather(n, d, chunk, nbuf)(idx, emb_table)
  return out.reshape(b, s, d)
